# baseline (speedup 1.0000x reference)
import functools

import jax
import jax.numpy as jnp
from jax import lax
from jax.experimental import pallas as pl
from jax.experimental.pallas import tpu as pltpu
from jax.experimental.pallas import tpu_sc as plsc

N = 10000
M = 5000
E = 320000
D = 128
NUM_LAYERS = 4

NC = 2
NS = 16
NW = NC * NS
EW = E // NW
C = 80
NCH = EW // C
M_PAD = 5120
N_PAD = 10240
CW = 16

_MESH = plsc.VectorSubcoreMesh(core_axis_name="c", subcore_axis_name="s")


def _fill_rows(buf, nrows, value):
    vec = jnp.full((16,), value, jnp.float32)

    @pl.loop(0, nrows)
    def _(r):
        @pl.loop(0, D // 16)
        def _(c16):
            buf[r, pl.ds(c16 * 16, 16)] = vec


def _stage_chunk(dst, src1d, base):
    @pl.loop(0, C // 16)
    def _(j):
        dst[pl.ds(j * 16, 16)] = src1d[pl.ds(base + j * 16, 16)]


_RPT = N_PAD // NS
EC = E // NS
NCH2 = EC // C


@functools.partial(
    pl.kernel,
    out_type=jax.ShapeDtypeStruct((NC, N_PAD, CW), jnp.float32),
    mesh=_MESH,
    scratch_types=[
        pltpu.VMEM((EC,), jnp.int32),
        pltpu.VMEM((C,), jnp.int32),
        pltpu.VMEM((C,), jnp.int32),
        pltpu.VMEM((C, CW), jnp.float32),
        pltpu.VMEM((C, CW), jnp.float32),
        pltpu.VMEM_SHARED((N_PAD, CW), jnp.float32),
        pltpu.SemaphoreType.DMA,
        pltpu.SemaphoreType.DMA,
    ],
)
def _seg_counts(idx_hbm, konst_hbm, out_hbm,
                idx_v, cbuf_a, cbuf_b, ones16, zbuf, acc, sem_a, sem_b):
    cid = lax.axis_index("c")
    sid = lax.axis_index("s")
    wid = cid * NS + sid
    pltpu.sync_copy(idx_hbm.at[wid], idx_v)
    pltpu.sync_copy(konst_hbm.at[0], ones16)
    pltpu.sync_copy(konst_hbm.at[1], zbuf)
    base = sid * _RPT

    @pl.loop(0, _RPT // C)
    def _(z):
        pltpu.sync_copy(zbuf, acc.at[pl.ds(base + z * C, C)])

    plsc.subcore_barrier()

    def start(cb, sem, kk):
        _stage_chunk(cb, idx_v, kk * C)
        pltpu.make_async_copy(ones16, acc.at[cb], sem).start(add=True)

    def finish(cb, sem):
        pltpu.make_async_copy(ones16, acc.at[cb], sem).wait()

    start(cbuf_a, sem_a, 0)

    @pl.loop(0, NCH2 // 2 - 1)
    def _(i):
        k0 = 2 * i
        start(cbuf_b, sem_b, k0 + 1)
        finish(cbuf_a, sem_a)
        start(cbuf_a, sem_a, k0 + 2)
        finish(cbuf_b, sem_b)

    start(cbuf_b, sem_b, NCH2 - 1)
    finish(cbuf_a, sem_a)
    finish(cbuf_b, sem_b)

    plsc.subcore_barrier()

    @pl.loop(0, _RPT // C)
    def _(z):
        pltpu.sync_copy(acc.at[pl.ds(base + z * C, C)], zbuf)
        pltpu.sync_copy(zbuf, out_hbm.at[cid, pl.ds(base + z * C, C)])


@functools.partial(
    pl.kernel,
    out_type=jax.ShapeDtypeStruct((NC, N_PAD, D), jnp.float32),
    mesh=_MESH,
    scratch_types=[
        pltpu.VMEM((EW,), jnp.int32),
        pltpu.VMEM((EW,), jnp.int32),
        pltpu.VMEM((C,), jnp.int32),
        pltpu.VMEM((C,), jnp.int32),
        pltpu.VMEM((C,), jnp.int32),
        pltpu.VMEM((C,), jnp.int32),
        pltpu.VMEM((C, D), jnp.float32),
        pltpu.VMEM((C, D), jnp.float32),
        pltpu.SemaphoreType.DMA,
        pltpu.SemaphoreType.DMA,
        pltpu.SemaphoreType.DMA,
        pltpu.SemaphoreType.DMA,
        pltpu.VMEM_SHARED((N_PAD, D), jnp.float32),
    ],
)
def _seg_sum(table_hbm, gidx_hbm, sidx_hbm, out_hbm,
             gidx_v, sidx_v, gi_a, gi_b, si_a, si_b,
             rows_a, rows_b,
             sg_a, sg_b, ss_a, ss_b, acc):
    cid = lax.axis_index("c")
    sid = lax.axis_index("s")
    wid = cid * NS + sid
    pltpu.sync_copy(gidx_hbm.at[wid], gidx_v)
    pltpu.sync_copy(sidx_hbm.at[wid], sidx_v)
    _fill_rows(rows_a, C, 0.0)
    base = sid * _RPT

    @pl.loop(0, _RPT // C)
    def _(z):
        pltpu.sync_copy(rows_a, acc.at[pl.ds(base + z * C, C)])

    plsc.subcore_barrier()

    def sg(gi, rows, sem, kk):
        _stage_chunk(gi, gidx_v, kk * C)
        pltpu.make_async_copy(table_hbm.at[gi], rows, sem).start()

    def sc(gi, si, rows, semg, sems, kk):
        pltpu.make_async_copy(table_hbm.at[gi], rows, semg).wait()
        _stage_chunk(si, sidx_v, kk * C)
        pltpu.make_async_copy(rows, acc.at[si], sems).start(add=True)

    def ws(si, rows, sems):
        pltpu.make_async_copy(rows, acc.at[si], sems).wait()

    sg(gi_a, rows_a, sg_a, 0)
    sg(gi_b, rows_b, sg_b, 1)

    @pl.loop(0, (NCH - 3) // 2)
    def _(i):
        k = 2 * i
        sc(gi_a, si_a, rows_a, sg_a, ss_a, k)
        sc(gi_b, si_b, rows_b, sg_b, ss_b, k + 1)
        ws(si_a, rows_a, ss_a)
        sg(gi_a, rows_a, sg_a, k + 2)
        ws(si_b, rows_b, ss_b)
        sg(gi_b, rows_b, sg_b, k + 3)

    sc(gi_a, si_a, rows_a, sg_a, ss_a, NCH - 3)
    sc(gi_b, si_b, rows_b, sg_b, ss_b, NCH - 2)
    ws(si_a, rows_a, ss_a)
    sg(gi_a, rows_a, sg_a, NCH - 1)
    ws(si_b, rows_b, ss_b)
    sc(gi_a, si_a, rows_a, sg_a, ss_a, NCH - 1)
    ws(si_a, rows_a, ss_a)

    plsc.subcore_barrier()

    @pl.loop(0, _RPT // C)
    def _(z):
        pltpu.sync_copy(acc.at[pl.ds(base + z * C, C)], rows_a)
        pltpu.sync_copy(rows_a, out_hbm.at[cid, pl.ds(base + z * C, C)])


_RB = 1000


def _enc_body(x_ref, we_ref, be_ref, w0_ref, b0_ref, o_ref):
    t = jnp.dot(x_ref[...], we_ref[...],
                preferred_element_type=jnp.float32) + be_ref[...]
    o_ref[...] = jnp.dot(t, w0_ref[...],
                         preferred_element_type=jnp.float32) + b0_ref[...]


def _encoder(x, W_enc, b_enc, W0, b0):
    return pl.pallas_call(
        _enc_body,
        grid=(N // _RB,),
        in_specs=[
            pl.BlockSpec((_RB, D), lambda i: (i, 0)),
            pl.BlockSpec((D, D), lambda i: (0, 0)),
            pl.BlockSpec((1, D), lambda i: (0, 0)),
            pl.BlockSpec((D, D), lambda i: (0, 0)),
            pl.BlockSpec((1, D), lambda i: (0, 0)),
        ],
        out_specs=pl.BlockSpec((_RB, D), lambda i: (i, 0)),
        out_shape=jax.ShapeDtypeStruct((N_PAD, D), jnp.float32),
    )(x, W_enc, b_enc.reshape(1, D), W0, b0.reshape(1, D))


def _ecomb_body(p_ref, c_ref, o_ref):
    cnt = c_ref[:, 0:1]
    inv = 1.0 / jnp.maximum(cnt, 1.0)
    o_ref[...] = (p_ref[0] + p_ref[1]) * inv


def _e_combine(p, cnt_e):
    blk = 1024
    return pl.pallas_call(
        _ecomb_body,
        grid=(M_PAD // blk,),
        in_specs=[
            pl.BlockSpec((NC, blk, D), lambda i: (0, i, 0)),
            pl.BlockSpec((blk, CW), lambda i: (i, 0)),
        ],
        out_specs=pl.BlockSpec((blk, D), lambda i: (i, 0)),
        out_shape=jax.ShapeDtypeStruct((N_PAD, D), jnp.float32),
    )(p, cnt_e)


def _layer_norm_relu(h, g, be):
    mu = jnp.mean(h, axis=-1, keepdims=True)
    d = h - mu
    var = jnp.mean(d * d, axis=-1, keepdims=True)
    t = g * d * lax.rsqrt(var + 1e-5) + be
    return jnp.maximum(t, 0.0)


def _make_update_body(first):
    def body(h_ref, q_ref, c_ref, g_ref, be_ref, w_ref, b_ref,
             h_out, x_out):
        cnt = c_ref[:, 0:1]
        inv = 1.0 / jnp.maximum(cnt, 1.0)
        r = jnp.maximum((q_ref[0] + q_ref[1]) * inv, 0.0)
        h = r if first else h_ref[...] + r
        h_out[...] = h
        t = _layer_norm_relu(h, g_ref[...], be_ref[...])
        x_out[...] = jnp.dot(t, w_ref[...],
                             preferred_element_type=jnp.float32) + b_ref[...]
    return body


def _layer_update(h, q, cnt_v, g, be, W, b, first):
    return pl.pallas_call(
        _make_update_body(first),
        grid=(N // _RB,),
        in_specs=[
            pl.BlockSpec((_RB, D), lambda i: (i, 0)),
            pl.BlockSpec((NC, _RB, D), lambda i: (0, i, 0)),
            pl.BlockSpec((_RB, CW), lambda i: (i, 0)),
            pl.BlockSpec((1, D), lambda i: (0, 0)),
            pl.BlockSpec((1, D), lambda i: (0, 0)),
            pl.BlockSpec((D, D), lambda i: (0, 0)),
            pl.BlockSpec((1, D), lambda i: (0, 0)),
        ],
        out_specs=(pl.BlockSpec((_RB, D), lambda i: (i, 0)),
                   pl.BlockSpec((_RB, D), lambda i: (i, 0))),
        out_shape=(jax.ShapeDtypeStruct((N, D), jnp.float32),
                   jax.ShapeDtypeStruct((N_PAD, D), jnp.float32)),
    )(h, q, cnt_v, g.reshape(1, D), be.reshape(1, D), W, b.reshape(1, D))


def kernel(x, vertex_idx, hyperedge_idx, W_enc, b_enc,
           W0, b0, g0, be0, W1, b1, g1, be1,
           W2, b2, g2, be2, W3, b3, g3, be3,
           W_lin, b_lin):
    gs = [g0, g1, g2, g3]
    bes = [be0, be1, be2, be3]
    Ws = [W0, W1, W2, W3]
    bs = [b0, b1, b2, b3]

    vflat = vertex_idx.astype(jnp.int32)
    eflat = hyperedge_idx.astype(jnp.int32)
    vidx = vflat.reshape(NW, EW)
    eidx = eflat.reshape(NW, EW)

    idx2 = jnp.concatenate(
        [eflat.reshape(NS, EC), vflat.reshape(NS, EC)], axis=0)
    konst = jnp.stack([jnp.ones((C, CW), jnp.float32),
                       jnp.zeros((C, CW), jnp.float32)])
    cnts = _seg_counts(idx2, konst)
    cnt_e = cnts[0, :M_PAD]
    cnt_v = cnts[1]

    xin = _encoder(x, W_enc, b_enc, W0, b0)

    h = None
    for i in range(NUM_LAYERS):
        p = _seg_sum(xin, vidx, eidx)
        e_feat = _e_combine(p, cnt_e)
        q = _seg_sum(e_feat, eidx, vidx)
        if i < NUM_LAYERS - 1:
            g_n, be_n, W_n, b_n = gs[i + 1], bes[i + 1], Ws[i + 1], bs[i + 1]
        else:
            g_n, be_n, W_n, b_n = g0, be0, W_lin, b_lin
        if i == 0:
            h, xin = _layer_update(jnp.zeros((N, D), jnp.float32), q, cnt_v,
                                   g_n, be_n, W_n, b_n, first=True)
        else:
            h, xin = _layer_update(h, q, cnt_v, g_n, be_n, W_n, b_n,
                                   first=False)
    return xin[:N]


# speedup vs baseline: 1.4615x; 1.4615x over previous
import functools

import jax
import jax.numpy as jnp
from jax import lax
from jax.experimental import pallas as pl
from jax.experimental.pallas import tpu as pltpu
from jax.experimental.pallas import tpu_sc as plsc

N = 10000
M = 5000
E = 320000
D = 128
NUM_LAYERS = 4

NC = 2
NS = 16
NW = NC * NS
EW = E // NW
C = 80
NCH = EW // C
M_PAD = 5120
N_PAD = 10240
CW = 16

_MESH = plsc.VectorSubcoreMesh(core_axis_name="c", subcore_axis_name="s")


def _fill_rows(buf, nrows, value):
    vec = jnp.full((16,), value, jnp.float32)

    @pl.loop(0, nrows)
    def _(r):
        @pl.loop(0, D // 16)
        def _(c16):
            buf[r, pl.ds(c16 * 16, 16)] = vec


def _stage_chunk(dst, src1d, base):
    @pl.loop(0, C // 16)
    def _(j):
        dst[pl.ds(j * 16, 16)] = src1d[pl.ds(base + j * 16, 16)]


_RPT = N_PAD // NS
EC = E // NS
NCH2 = EC // C


@functools.partial(
    pl.kernel,
    out_type=jax.ShapeDtypeStruct((NC, N_PAD, CW), jnp.float32),
    mesh=_MESH,
    scratch_types=[
        pltpu.VMEM((EC,), jnp.int32),
        pltpu.VMEM((C,), jnp.int32),
        pltpu.VMEM((C,), jnp.int32),
        pltpu.VMEM((C, CW), jnp.float32),
        pltpu.VMEM((C, CW), jnp.float32),
        pltpu.VMEM_SHARED((N_PAD, CW), jnp.float32),
        pltpu.SemaphoreType.DMA,
        pltpu.SemaphoreType.DMA,
    ],
)
def _seg_counts(idx_hbm, konst_hbm, out_hbm,
                idx_v, cbuf_a, cbuf_b, ones16, zbuf, acc, sem_a, sem_b):
    cid = lax.axis_index("c")
    sid = lax.axis_index("s")
    wid = cid * NS + sid
    pltpu.sync_copy(idx_hbm.at[wid], idx_v)
    pltpu.sync_copy(konst_hbm.at[0], ones16)
    pltpu.sync_copy(konst_hbm.at[1], zbuf)
    base = sid * _RPT

    @pl.loop(0, _RPT // C)
    def _(z):
        pltpu.sync_copy(zbuf, acc.at[pl.ds(base + z * C, C)])

    plsc.subcore_barrier()

    def start(cb, sem, kk):
        _stage_chunk(cb, idx_v, kk * C)
        pltpu.make_async_copy(ones16, acc.at[cb], sem).start(add=True)

    def finish(cb, sem):
        pltpu.make_async_copy(ones16, acc.at[cb], sem).wait()

    start(cbuf_a, sem_a, 0)

    @pl.loop(0, NCH2 // 2 - 1)
    def _(i):
        k0 = 2 * i
        start(cbuf_b, sem_b, k0 + 1)
        finish(cbuf_a, sem_a)
        start(cbuf_a, sem_a, k0 + 2)
        finish(cbuf_b, sem_b)

    start(cbuf_b, sem_b, NCH2 - 1)
    finish(cbuf_a, sem_a)
    finish(cbuf_b, sem_b)

    plsc.subcore_barrier()

    @pl.loop(0, _RPT // C)
    def _(z):
        pltpu.sync_copy(acc.at[pl.ds(base + z * C, C)], zbuf)
        pltpu.sync_copy(zbuf, out_hbm.at[cid, pl.ds(base + z * C, C)])


@functools.partial(
    pl.kernel,
    out_type=jax.ShapeDtypeStruct((NC, N_PAD, D), jnp.float32),
    mesh=_MESH,
    scratch_types=[
        pltpu.VMEM((EW,), jnp.int32),
        pltpu.VMEM((C,), jnp.int32),
        pltpu.VMEM((C,), jnp.int32),
        pltpu.VMEM((C,), jnp.int32),
        pltpu.VMEM((C,), jnp.int32),
        pltpu.VMEM((C,), jnp.int32),
        pltpu.VMEM((C,), jnp.int32),
        pltpu.VMEM((C, D), jnp.float32),
        pltpu.VMEM((C, D), jnp.float32),
        pltpu.VMEM((C, D), jnp.float32),
        pltpu.SemaphoreType.DMA,
        pltpu.SemaphoreType.DMA,
        pltpu.SemaphoreType.DMA,
        pltpu.SemaphoreType.DMA,
        pltpu.SemaphoreType.DMA,
        pltpu.SemaphoreType.DMA,
        pltpu.VMEM_SHARED((N_PAD, D), jnp.float32),
    ],
)
def _seg_sum(table_hbm, pidx_hbm, out_hbm,
             pidx_v, gi_a, gi_b, gi_c, si_a, si_b, si_c,
             rows_a, rows_b, rows_c,
             sg_a, sg_b, sg_c, ss_a, ss_b, ss_c, acc):
    cid = lax.axis_index("c")
    sid = lax.axis_index("s")
    wid = cid * NS + sid
    pltpu.sync_copy(pidx_hbm.at[wid], pidx_v)
    _fill_rows(rows_a, C, 0.0)
    base = sid * _RPT

    @pl.loop(0, _RPT // C)
    def _(z):
        pltpu.sync_copy(rows_a, acc.at[pl.ds(base + z * C, C)])

    plsc.subcore_barrier()

    def sg(gi, si, rows, semg, kk):
        @pl.loop(0, C // 16)
        def _(j):
            v = pidx_v[pl.ds(kk * C + j * 16, 16)]
            gi[pl.ds(j * 16, 16)] = jnp.bitwise_and(v, jnp.int32(0xFFFF))
            si[pl.ds(j * 16, 16)] = jnp.right_shift(v, jnp.int32(16))

        pltpu.make_async_copy(table_hbm.at[gi], rows, semg).start()

    def sc(gi, si, rows, semg, sems):
        pltpu.make_async_copy(table_hbm.at[gi], rows, semg).wait()
        pltpu.make_async_copy(rows, acc.at[si], sems).start(add=True)

    def ws(si, rows, sems):
        pltpu.make_async_copy(rows, acc.at[si], sems).wait()

    sg(gi_a, si_a, rows_a, sg_a, 0)
    sg(gi_b, si_b, rows_b, sg_b, 1)
    sc(gi_a, si_a, rows_a, sg_a, ss_a)

    @pl.loop(0, (NCH - 2) // 3)
    def _(i):
        k = 3 * i
        sg(gi_c, si_c, rows_c, sg_c, k + 2)
        sc(gi_b, si_b, rows_b, sg_b, ss_b)
        ws(si_a, rows_a, ss_a)
        sg(gi_a, si_a, rows_a, sg_a, k + 3)
        sc(gi_c, si_c, rows_c, sg_c, ss_c)
        ws(si_b, rows_b, ss_b)
        sg(gi_b, si_b, rows_b, sg_b, k + 4)
        sc(gi_a, si_a, rows_a, sg_a, ss_a)
        ws(si_c, rows_c, ss_c)

    sc(gi_b, si_b, rows_b, sg_b, ss_b)
    ws(si_a, rows_a, ss_a)
    ws(si_b, rows_b, ss_b)

    plsc.subcore_barrier()

    @pl.loop(0, _RPT // C)
    def _(z):
        pltpu.sync_copy(acc.at[pl.ds(base + z * C, C)], rows_a)
        pltpu.sync_copy(rows_a, out_hbm.at[cid, pl.ds(base + z * C, C)])


_RB = 1000


def _enc_body(x_ref, we_ref, be_ref, w0_ref, b0_ref, o_ref):
    t = jnp.dot(x_ref[...], we_ref[...],
                preferred_element_type=jnp.float32) + be_ref[...]
    o_ref[...] = jnp.dot(t, w0_ref[...],
                         preferred_element_type=jnp.float32) + b0_ref[...]


def _encoder(x, W_enc, b_enc, W0, b0):
    return pl.pallas_call(
        _enc_body,
        grid=(N // _RB,),
        in_specs=[
            pl.BlockSpec((_RB, D), lambda i: (i, 0)),
            pl.BlockSpec((D, D), lambda i: (0, 0)),
            pl.BlockSpec((1, D), lambda i: (0, 0)),
            pl.BlockSpec((D, D), lambda i: (0, 0)),
            pl.BlockSpec((1, D), lambda i: (0, 0)),
        ],
        out_specs=pl.BlockSpec((_RB, D), lambda i: (i, 0)),
        out_shape=jax.ShapeDtypeStruct((N_PAD, D), jnp.float32),
    )(x, W_enc, b_enc.reshape(1, D), W0, b0.reshape(1, D))


def _ecomb_body(p_ref, c_ref, o_ref):
    cnt = c_ref[:, 0:1]
    inv = 1.0 / jnp.maximum(cnt, 1.0)
    o_ref[...] = (p_ref[0] + p_ref[1]) * inv


def _e_combine(p, cnt_e):
    blk = 1024
    return pl.pallas_call(
        _ecomb_body,
        grid=(M_PAD // blk,),
        in_specs=[
            pl.BlockSpec((NC, blk, D), lambda i: (0, i, 0)),
            pl.BlockSpec((blk, CW), lambda i: (i, 0)),
        ],
        out_specs=pl.BlockSpec((blk, D), lambda i: (i, 0)),
        out_shape=jax.ShapeDtypeStruct((N_PAD, D), jnp.float32),
    )(p, cnt_e)


def _layer_norm_relu(h, g, be):
    mu = jnp.mean(h, axis=-1, keepdims=True)
    d = h - mu
    var = jnp.mean(d * d, axis=-1, keepdims=True)
    t = g * d * lax.rsqrt(var + 1e-5) + be
    return jnp.maximum(t, 0.0)


def _make_update_body(first):
    def body(h_ref, q_ref, c_ref, g_ref, be_ref, w_ref, b_ref,
             h_out, x_out):
        cnt = c_ref[:, 0:1]
        inv = 1.0 / jnp.maximum(cnt, 1.0)
        r = jnp.maximum((q_ref[0] + q_ref[1]) * inv, 0.0)
        h = r if first else h_ref[...] + r
        h_out[...] = h
        t = _layer_norm_relu(h, g_ref[...], be_ref[...])
        x_out[...] = jnp.dot(t, w_ref[...],
                             preferred_element_type=jnp.float32) + b_ref[...]
    return body


def _layer_update(h, q, cnt_v, g, be, W, b, first):
    return pl.pallas_call(
        _make_update_body(first),
        grid=(N // _RB,),
        in_specs=[
            pl.BlockSpec((_RB, D), lambda i: (i, 0)),
            pl.BlockSpec((NC, _RB, D), lambda i: (0, i, 0)),
            pl.BlockSpec((_RB, CW), lambda i: (i, 0)),
            pl.BlockSpec((1, D), lambda i: (0, 0)),
            pl.BlockSpec((1, D), lambda i: (0, 0)),
            pl.BlockSpec((D, D), lambda i: (0, 0)),
            pl.BlockSpec((1, D), lambda i: (0, 0)),
        ],
        out_specs=(pl.BlockSpec((_RB, D), lambda i: (i, 0)),
                   pl.BlockSpec((_RB, D), lambda i: (i, 0))),
        out_shape=(jax.ShapeDtypeStruct((N, D), jnp.float32),
                   jax.ShapeDtypeStruct((N_PAD, D), jnp.float32)),
    )(h, q, cnt_v, g.reshape(1, D), be.reshape(1, D), W, b.reshape(1, D))


def kernel(x, vertex_idx, hyperedge_idx, W_enc, b_enc,
           W0, b0, g0, be0, W1, b1, g1, be1,
           W2, b2, g2, be2, W3, b3, g3, be3,
           W_lin, b_lin):
    gs = [g0, g1, g2, g3]
    bes = [be0, be1, be2, be3]
    Ws = [W0, W1, W2, W3]
    bs = [b0, b1, b2, b3]

    vflat = vertex_idx.astype(jnp.int32)
    eflat = hyperedge_idx.astype(jnp.int32)
    vidx = vflat.reshape(NW, EW)
    eidx = eflat.reshape(NW, EW)
    pidx_p = jnp.bitwise_or(vidx, jnp.left_shift(eidx, 16))
    pidx_q = jnp.bitwise_or(eidx, jnp.left_shift(vidx, 16))

    idx2 = jnp.concatenate(
        [eflat.reshape(NS, EC), vflat.reshape(NS, EC)], axis=0)
    konst = jnp.stack([jnp.ones((C, CW), jnp.float32),
                       jnp.zeros((C, CW), jnp.float32)])
    cnts = _seg_counts(idx2, konst)
    cnt_e = cnts[0, :M_PAD]
    cnt_v = cnts[1]

    xin = _encoder(x, W_enc, b_enc, W0, b0)

    h = None
    for i in range(NUM_LAYERS):
        p = _seg_sum(xin, pidx_p)
        e_feat = _e_combine(p, cnt_e)
        q = _seg_sum(e_feat, pidx_q)
        if i < NUM_LAYERS - 1:
            g_n, be_n, W_n, b_n = gs[i + 1], bes[i + 1], Ws[i + 1], bs[i + 1]
        else:
            g_n, be_n, W_n, b_n = g0, be0, W_lin, b_lin
        if i == 0:
            h, xin = _layer_update(jnp.zeros((N, D), jnp.float32), q, cnt_v,
                                   g_n, be_n, W_n, b_n, first=True)
        else:
            h, xin = _layer_update(h, q, cnt_v, g_n, be_n, W_n, b_n,
                                   first=False)
    return xin[:N]


# speedup vs baseline: 1.4734x; 1.0082x over previous
import functools

import jax
import jax.numpy as jnp
from jax import lax
from jax.experimental import pallas as pl
from jax.experimental.pallas import tpu as pltpu
from jax.experimental.pallas import tpu_sc as plsc

N = 10000
M = 5000
E = 320000
D = 128
NUM_LAYERS = 4

NC = 2
NS = 16
NW = NC * NS
EW = E // NW
C = 80
NCH = EW // C
M_PAD = 5120
N_PAD = 10240
CW = 16

_MESH = plsc.VectorSubcoreMesh(core_axis_name="c", subcore_axis_name="s")


def _fill_rows(buf, nrows, value):
    vec = jnp.full((16,), value, jnp.float32)

    @pl.loop(0, nrows)
    def _(r):
        @pl.loop(0, D // 16)
        def _(c16):
            buf[r, pl.ds(c16 * 16, 16)] = vec


def _stage_chunk(dst, src1d, base):
    @pl.loop(0, C // 16)
    def _(j):
        dst[pl.ds(j * 16, 16)] = src1d[pl.ds(base + j * 16, 16)]


_RPT = N_PAD // NS
EC = E // NS
NCH2 = EC // C


@functools.partial(
    pl.kernel,
    out_type=jax.ShapeDtypeStruct((NC, N_PAD, CW), jnp.float32),
    mesh=_MESH,
    scratch_types=[
        pltpu.VMEM((EC,), jnp.int32),
        pltpu.VMEM((C,), jnp.int32),
        pltpu.VMEM((C,), jnp.int32),
        pltpu.VMEM((C, CW), jnp.float32),
        pltpu.VMEM((C, CW), jnp.float32),
        pltpu.VMEM_SHARED((N_PAD, CW), jnp.float32),
        pltpu.SemaphoreType.DMA,
        pltpu.SemaphoreType.DMA,
    ],
)
def _seg_counts(idx_hbm, konst_hbm, out_hbm,
                idx_v, cbuf_a, cbuf_b, ones16, zbuf, acc, sem_a, sem_b):
    cid = lax.axis_index("c")
    sid = lax.axis_index("s")
    wid = cid * NS + sid
    pltpu.sync_copy(idx_hbm.at[wid], idx_v)
    pltpu.sync_copy(konst_hbm.at[0], ones16)
    pltpu.sync_copy(konst_hbm.at[1], zbuf)
    base = sid * _RPT

    @pl.loop(0, _RPT // C)
    def _(z):
        pltpu.sync_copy(zbuf, acc.at[pl.ds(base + z * C, C)])

    plsc.subcore_barrier()

    def start(cb, sem, kk):
        _stage_chunk(cb, idx_v, kk * C)
        pltpu.make_async_copy(ones16, acc.at[cb], sem).start(add=True)

    def finish(cb, sem):
        pltpu.make_async_copy(ones16, acc.at[cb], sem).wait()

    start(cbuf_a, sem_a, 0)

    @pl.loop(0, NCH2 // 2 - 1)
    def _(i):
        k0 = 2 * i
        start(cbuf_b, sem_b, k0 + 1)
        finish(cbuf_a, sem_a)
        start(cbuf_a, sem_a, k0 + 2)
        finish(cbuf_b, sem_b)

    start(cbuf_b, sem_b, NCH2 - 1)
    finish(cbuf_a, sem_a)
    finish(cbuf_b, sem_b)

    plsc.subcore_barrier()

    @pl.loop(0, _RPT // C)
    def _(z):
        pltpu.sync_copy(acc.at[pl.ds(base + z * C, C)], zbuf)
        pltpu.sync_copy(zbuf, out_hbm.at[cid, pl.ds(base + z * C, C)])


@functools.partial(
    pl.kernel,
    out_type=jax.ShapeDtypeStruct((NC, N_PAD, D), jnp.float32),
    mesh=_MESH,
    scratch_types=[
        pltpu.VMEM((EW,), jnp.int32),
        pltpu.VMEM((C,), jnp.int32),
        pltpu.VMEM((C,), jnp.int32),
        pltpu.VMEM((C,), jnp.int32),
        pltpu.VMEM((C,), jnp.int32),
        pltpu.VMEM((C,), jnp.int32),
        pltpu.VMEM((C,), jnp.int32),
        pltpu.VMEM((C, D), jnp.float32),
        pltpu.VMEM((C, D), jnp.float32),
        pltpu.VMEM((C, D), jnp.float32),
        pltpu.SemaphoreType.DMA,
        pltpu.SemaphoreType.DMA,
        pltpu.SemaphoreType.DMA,
        pltpu.SemaphoreType.DMA,
        pltpu.SemaphoreType.DMA,
        pltpu.SemaphoreType.DMA,
        pltpu.VMEM_SHARED((N_PAD, D), jnp.float32),
    ],
)
def _seg_sum(table_hbm, pidx_hbm, out_hbm,
             pidx_v, gi_a, gi_b, gi_c, si_a, si_b, si_c,
             rows_a, rows_b, rows_c,
             sg_a, sg_b, sg_c, ss_a, ss_b, ss_c, acc):
    cid = lax.axis_index("c")
    sid = lax.axis_index("s")
    wid = cid * NS + sid
    pltpu.sync_copy(pidx_hbm.at[wid], pidx_v)
    _fill_rows(rows_a, C, 0.0)
    base = sid * _RPT

    @pl.loop(0, _RPT // C)
    def _(z):
        pltpu.make_async_copy(rows_a, acc.at[pl.ds(base + z * C, C)],
                              sg_a).start()

    @pl.loop(0, _RPT // C)
    def _(z):
        pltpu.make_async_copy(rows_a, acc.at[pl.ds(base + z * C, C)],
                              sg_a).wait()

    plsc.subcore_barrier()

    def sg(gi, si, rows, semg, kk):
        @pl.loop(0, C // 16)
        def _(j):
            v = pidx_v[pl.ds(kk * C + j * 16, 16)]
            gi[pl.ds(j * 16, 16)] = jnp.bitwise_and(v, jnp.int32(0xFFFF))
            si[pl.ds(j * 16, 16)] = jnp.right_shift(v, jnp.int32(16))

        pltpu.make_async_copy(table_hbm.at[gi], rows, semg).start()

    def sc(gi, si, rows, semg, sems):
        pltpu.make_async_copy(table_hbm.at[gi], rows, semg).wait()
        pltpu.make_async_copy(rows, acc.at[si], sems).start(add=True)

    def ws(si, rows, sems):
        pltpu.make_async_copy(rows, acc.at[si], sems).wait()

    sg(gi_a, si_a, rows_a, sg_a, 0)
    sg(gi_b, si_b, rows_b, sg_b, 1)
    sc(gi_a, si_a, rows_a, sg_a, ss_a)

    @pl.loop(0, (NCH - 2) // 3)
    def _(i):
        k = 3 * i
        sg(gi_c, si_c, rows_c, sg_c, k + 2)
        sc(gi_b, si_b, rows_b, sg_b, ss_b)
        ws(si_a, rows_a, ss_a)
        sg(gi_a, si_a, rows_a, sg_a, k + 3)
        sc(gi_c, si_c, rows_c, sg_c, ss_c)
        ws(si_b, rows_b, ss_b)
        sg(gi_b, si_b, rows_b, sg_b, k + 4)
        sc(gi_a, si_a, rows_a, sg_a, ss_a)
        ws(si_c, rows_c, ss_c)

    sc(gi_b, si_b, rows_b, sg_b, ss_b)
    ws(si_a, rows_a, ss_a)
    ws(si_b, rows_b, ss_b)

    plsc.subcore_barrier()

    def rd(rows, sem, z):
        pltpu.make_async_copy(acc.at[pl.ds(base + z * C, C)], rows,
                              sem).start()

    def rdw(rows, sem, z):
        pltpu.make_async_copy(acc.at[pl.ds(base + z * C, C)], rows,
                              sem).wait()

    def wr(rows, sem, z):
        pltpu.make_async_copy(rows, out_hbm.at[cid, pl.ds(base + z * C, C)],
                              sem).start()

    def wrw(rows, sem, z):
        pltpu.make_async_copy(rows, out_hbm.at[cid, pl.ds(base + z * C, C)],
                              sem).wait()

    rd(rows_a, sg_a, 0)
    rd(rows_b, sg_b, 1)

    @pl.loop(0, _RPT // C // 2 - 1)
    def _(i):
        z = 2 * i
        rdw(rows_a, sg_a, z)
        wr(rows_a, ss_a, z)
        rdw(rows_b, sg_b, z + 1)
        wr(rows_b, ss_b, z + 1)
        wrw(rows_a, ss_a, z)
        rd(rows_a, sg_a, z + 2)
        wrw(rows_b, ss_b, z + 1)
        rd(rows_b, sg_b, z + 3)

    zl = _RPT // C - 2
    rdw(rows_a, sg_a, zl)
    wr(rows_a, ss_a, zl)
    rdw(rows_b, sg_b, zl + 1)
    wr(rows_b, ss_b, zl + 1)
    wrw(rows_a, ss_a, zl)
    wrw(rows_b, ss_b, zl + 1)


_RB = 1000


def _enc_body(x_ref, we_ref, be_ref, w0_ref, b0_ref, o_ref):
    t = jnp.dot(x_ref[...], we_ref[...],
                preferred_element_type=jnp.float32) + be_ref[...]
    o_ref[...] = jnp.dot(t, w0_ref[...],
                         preferred_element_type=jnp.float32) + b0_ref[...]


def _encoder(x, W_enc, b_enc, W0, b0):
    return pl.pallas_call(
        _enc_body,
        grid=(N // _RB,),
        in_specs=[
            pl.BlockSpec((_RB, D), lambda i: (i, 0)),
            pl.BlockSpec((D, D), lambda i: (0, 0)),
            pl.BlockSpec((1, D), lambda i: (0, 0)),
            pl.BlockSpec((D, D), lambda i: (0, 0)),
            pl.BlockSpec((1, D), lambda i: (0, 0)),
        ],
        out_specs=pl.BlockSpec((_RB, D), lambda i: (i, 0)),
        out_shape=jax.ShapeDtypeStruct((N_PAD, D), jnp.float32),
    )(x, W_enc, b_enc.reshape(1, D), W0, b0.reshape(1, D))


def _ecomb_body(p_ref, c_ref, o_ref):
    cnt = c_ref[:, 0:1]
    inv = 1.0 / jnp.maximum(cnt, 1.0)
    o_ref[...] = (p_ref[0] + p_ref[1]) * inv


def _e_combine(p, cnt_e):
    blk = 1024
    return pl.pallas_call(
        _ecomb_body,
        grid=(M_PAD // blk,),
        in_specs=[
            pl.BlockSpec((NC, blk, D), lambda i: (0, i, 0)),
            pl.BlockSpec((blk, CW), lambda i: (i, 0)),
        ],
        out_specs=pl.BlockSpec((blk, D), lambda i: (i, 0)),
        out_shape=jax.ShapeDtypeStruct((N_PAD, D), jnp.float32),
    )(p, cnt_e)


def _layer_norm_relu(h, g, be):
    mu = jnp.mean(h, axis=-1, keepdims=True)
    d = h - mu
    var = jnp.mean(d * d, axis=-1, keepdims=True)
    t = g * d * lax.rsqrt(var + 1e-5) + be
    return jnp.maximum(t, 0.0)


def _make_update_body(first):
    def body(h_ref, q_ref, c_ref, g_ref, be_ref, w_ref, b_ref,
             h_out, x_out):
        cnt = c_ref[:, 0:1]
        inv = 1.0 / jnp.maximum(cnt, 1.0)
        r = jnp.maximum((q_ref[0] + q_ref[1]) * inv, 0.0)
        h = r if first else h_ref[...] + r
        h_out[...] = h
        t = _layer_norm_relu(h, g_ref[...], be_ref[...])
        x_out[...] = jnp.dot(t, w_ref[...],
                             preferred_element_type=jnp.float32) + b_ref[...]
    return body


def _layer_update(h, q, cnt_v, g, be, W, b, first):
    return pl.pallas_call(
        _make_update_body(first),
        grid=(N // _RB,),
        in_specs=[
            pl.BlockSpec((_RB, D), lambda i: (i, 0)),
            pl.BlockSpec((NC, _RB, D), lambda i: (0, i, 0)),
            pl.BlockSpec((_RB, CW), lambda i: (i, 0)),
            pl.BlockSpec((1, D), lambda i: (0, 0)),
            pl.BlockSpec((1, D), lambda i: (0, 0)),
            pl.BlockSpec((D, D), lambda i: (0, 0)),
            pl.BlockSpec((1, D), lambda i: (0, 0)),
        ],
        out_specs=(pl.BlockSpec((_RB, D), lambda i: (i, 0)),
                   pl.BlockSpec((_RB, D), lambda i: (i, 0))),
        out_shape=(jax.ShapeDtypeStruct((N, D), jnp.float32),
                   jax.ShapeDtypeStruct((N_PAD, D), jnp.float32)),
    )(h, q, cnt_v, g.reshape(1, D), be.reshape(1, D), W, b.reshape(1, D))


def kernel(x, vertex_idx, hyperedge_idx, W_enc, b_enc,
           W0, b0, g0, be0, W1, b1, g1, be1,
           W2, b2, g2, be2, W3, b3, g3, be3,
           W_lin, b_lin):
    gs = [g0, g1, g2, g3]
    bes = [be0, be1, be2, be3]
    Ws = [W0, W1, W2, W3]
    bs = [b0, b1, b2, b3]

    vflat = vertex_idx.astype(jnp.int32)
    eflat = hyperedge_idx.astype(jnp.int32)
    vidx = vflat.reshape(NW, EW)
    eidx = eflat.reshape(NW, EW)
    pidx_p = jnp.bitwise_or(vidx, jnp.left_shift(eidx, 16))
    pidx_q = jnp.bitwise_or(eidx, jnp.left_shift(vidx, 16))

    idx2 = jnp.concatenate(
        [eflat.reshape(NS, EC), vflat.reshape(NS, EC)], axis=0)
    konst = jnp.stack([jnp.ones((C, CW), jnp.float32),
                       jnp.zeros((C, CW), jnp.float32)])
    cnts = _seg_counts(idx2, konst)
    cnt_e = cnts[0, :M_PAD]
    cnt_v = cnts[1]

    xin = _encoder(x, W_enc, b_enc, W0, b0)

    h = None
    for i in range(NUM_LAYERS):
        p = _seg_sum(xin, pidx_p)
        e_feat = _e_combine(p, cnt_e)
        q = _seg_sum(e_feat, pidx_q)
        if i < NUM_LAYERS - 1:
            g_n, be_n, W_n, b_n = gs[i + 1], bes[i + 1], Ws[i + 1], bs[i + 1]
        else:
            g_n, be_n, W_n, b_n = g0, be0, W_lin, b_lin
        if i == 0:
            h, xin = _layer_update(jnp.zeros((N, D), jnp.float32), q, cnt_v,
                                   g_n, be_n, W_n, b_n, first=True)
        else:
            h, xin = _layer_update(h, q, cnt_v, g_n, be_n, W_n, b_n,
                                   first=False)
    return xin[:N]
